# Initial kernel scaffold; baseline (speedup 1.0000x reference)
import jax
import jax.numpy as jnp
from jax.experimental import pallas as pl


def kernel(group_indices, embedding):
    raise NotImplementedError("write your pallas kernel here")


# speedup vs baseline: 1.5479x; 1.5479x over previous
import functools

import jax
import jax.numpy as jnp
from jax import lax
from jax.experimental import pallas as pl
from jax.experimental.pallas import tpu as pltpu
from jax.experimental.pallas import tpu_sc as plsc

FEATURES_DIM = 32
BATCH = 16384
N_FIELDS = 26
B = BATCH * N_FIELDS
NUM_CORES = 2
NUM_SUBCORES = 16
NUM_WORKERS = NUM_CORES * NUM_SUBCORES
B_PER_W = B // NUM_WORKERS
CHUNK = 1024
N_CHUNKS = B_PER_W // CHUNK

_mesh = plsc.VectorSubcoreMesh(core_axis_name="c", subcore_axis_name="s")


@functools.partial(
    pl.kernel,
    mesh=_mesh,
    out_type=jax.ShapeDtypeStruct((B, FEATURES_DIM), jnp.float32),
    scratch_types=[
        pltpu.VMEM((CHUNK,), jnp.int32),
        pltpu.VMEM((CHUNK, FEATURES_DIM), jnp.float32),
        pltpu.SemaphoreType.DMA,
    ],
    compiler_params=pltpu.CompilerParams(use_tc_tiling_on_sc=False),
)
def _gather_kernel(idx_hbm, table_hbm, out_hbm, idx_v, rows_v, sem):
    wid = lax.axis_index("s") * NUM_CORES + lax.axis_index("c")
    base = wid * B_PER_W

    def body(j, carry):
        off = base + j * CHUNK
        pltpu.sync_copy(idx_hbm.at[pl.ds(off, CHUNK)], idx_v)
        pltpu.async_copy(table_hbm.at[idx_v], rows_v, sem).wait()
        pltpu.sync_copy(rows_v, out_hbm.at[pl.ds(off, CHUNK)])
        return carry

    lax.fori_loop(0, N_CHUNKS, body, 0)


def kernel(group_indices, embedding):
    idx = group_indices.reshape(-1).astype(jnp.int32)
    out = _gather_kernel(idx, embedding)
    return out.reshape(group_indices.shape + (FEATURES_DIM,))


# speedup vs baseline: 1.5610x; 1.0084x over previous
import functools

import jax
import jax.numpy as jnp
from jax import lax
from jax.experimental import pallas as pl
from jax.experimental.pallas import tpu as pltpu
from jax.experimental.pallas import tpu_sc as plsc

FEATURES_DIM = 32
BATCH = 16384
N_FIELDS = 26
B = BATCH * N_FIELDS
NUM_CORES = 2
NUM_SUBCORES = 16
NUM_WORKERS = NUM_CORES * NUM_SUBCORES
B_PER_W = B // NUM_WORKERS
CHUNK = 1024
N_CHUNKS = B_PER_W // CHUNK
NBUF = 3

_mesh = plsc.VectorSubcoreMesh(core_axis_name="c", subcore_axis_name="s")


@functools.partial(
    pl.kernel,
    mesh=_mesh,
    out_type=jax.ShapeDtypeStruct((B, FEATURES_DIM), jnp.float32),
    scratch_types=[
        [pltpu.VMEM((CHUNK,), jnp.int32)] * N_CHUNKS,
        [pltpu.VMEM((CHUNK, FEATURES_DIM), jnp.float32)] * NBUF,
        [pltpu.SemaphoreType.DMA] * NBUF,
        [pltpu.SemaphoreType.DMA] * NBUF,
    ],
    compiler_params=pltpu.CompilerParams(use_tc_tiling_on_sc=False),
)
def _gather_kernel(idx_hbm, table_hbm, out_hbm, idx_vs, rows_v, gsems, osems):
    wid = lax.axis_index("s") * NUM_CORES + lax.axis_index("c")
    base = wid * B_PER_W

    for j in range(N_CHUNKS):
        pltpu.sync_copy(idx_hbm.at[pl.ds(base + j * CHUNK, CHUNK)], idx_vs[j])

    gh = [None] * N_CHUNKS

    def start_gather(j):
        b = j % NBUF
        gh[j] = pltpu.async_copy(
            table_hbm.at[idx_vs[j]], rows_v[b], gsems[b])

    for j in range(min(NBUF, N_CHUNKS)):
        start_gather(j)
    for j in range(N_CHUNKS):
        b = j % NBUF
        gh[j].wait()
        pltpu.sync_copy(rows_v[b], out_hbm.at[pl.ds(base + j * CHUNK, CHUNK)])
        nxt = j + NBUF
        if nxt < N_CHUNKS:
            start_gather(nxt)


def kernel(group_indices, embedding):
    idx = group_indices.reshape(-1).astype(jnp.int32)
    out = _gather_kernel(idx, embedding)
    return out.reshape(group_indices.shape + (FEATURES_DIM,))


# speedup vs baseline: 1.9651x; 1.2589x over previous
import functools

import jax
import jax.numpy as jnp
from jax import lax
from jax.experimental import pallas as pl
from jax.experimental.pallas import tpu as pltpu
from jax.experimental.pallas import tpu_sc as plsc

FEATURES_DIM = 32
BATCH = 16384
N_FIELDS = 26
NUM_CORES = 2
NUM_SUBCORES = 16
NUM_WORKERS = NUM_CORES * NUM_SUBCORES
BB = 64
NBLK = BATCH // BB
BLK_PER_W = NBLK // NUM_WORKERS
CHUNK = BB * N_FIELDS
NBUF = 2

_mesh = plsc.VectorSubcoreMesh(core_axis_name="c", subcore_axis_name="s")


@functools.partial(
    pl.kernel,
    mesh=_mesh,
    out_type=jax.ShapeDtypeStruct((BATCH, 32, 128), jnp.float32),
    scratch_types=[
        [pltpu.VMEM((CHUNK,), jnp.int32)] * NBUF,
        [pltpu.VMEM((CHUNK, FEATURES_DIM), jnp.float32)] * NBUF,
        [pltpu.SemaphoreType.DMA] * NBUF,
        [pltpu.SemaphoreType.DMA] * NBUF,
    ],
    compiler_params=pltpu.CompilerParams(use_tc_tiling_on_sc=False),
)
def _gather_kernel(idx_hbm, table_hbm, out_hbm, idx_vs, rows_v, gsems, wsems):
    wid = lax.axis_index("s") * NUM_CORES + lax.axis_index("c")
    base = wid * BLK_PER_W

    gh = [None] * BLK_PER_W

    def start_gather(j):
        b = j % NBUF
        pltpu.sync_copy(idx_hbm.at[pl.ds((base + j) * CHUNK, CHUNK)], idx_vs[b])
        gh[j] = pltpu.async_copy(
            table_hbm.at[idx_vs[b]], rows_v[b], gsems[b])

    for j in range(NBUF):
        start_gather(j)
    for j in range(BLK_PER_W):
        b = j % NBUF
        gh[j].wait()
        b0 = (base + j) * BB
        whs = [
            pltpu.async_copy(
                rows_v[b].at[pl.ds(bb * N_FIELDS, N_FIELDS), :],
                out_hbm.at[b0 + bb, pl.ds(0, N_FIELDS),
                           pl.ds(0, FEATURES_DIM)],
                wsems[b])
            for bb in range(BB)
        ]
        for w in whs:
            w.wait()
        nxt = j + NBUF
        if nxt < BLK_PER_W:
            start_gather(nxt)


def kernel(group_indices, embedding):
    idx = group_indices.reshape(-1).astype(jnp.int32)
    y = _gather_kernel(idx, embedding)
    return y[:, :N_FIELDS, :FEATURES_DIM]
